# Initial kernel scaffold; baseline (speedup 1.0000x reference)
import jax
import jax.numpy as jnp
from jax.experimental import pallas as pl


def kernel(x, edge_index, edge_attr, batch, W1, b1, s1_Wr, s1_br, s1_Ws, W2, b2, s2_Wr, s2_br, s2_Ws, W3, b3, s3_Wr, s3_br, s3_Ws, lstm_Wih0, lstm_Whh0, lstm_b0, lstm_Wih1, lstm_Whh1, lstm_b1, M1, mb1, M2, mb2, M3, mb3):
    raise NotImplementedError("write your pallas kernel here")


# speedup vs baseline: 1.0416x; 1.0416x over previous
import functools
import jax
import jax.numpy as jnp
from jax.experimental import pallas as pl
from jax.experimental.pallas import tpu as pltpu

_B = 10
_N0 = 2000
_N = _B * _N0
_E = 320000
_D = 128
_H = 100


def _mm_body(x_ref, w_ref, b_ref, o_ref):
    o_ref[...] = (
        jnp.dot(x_ref[...], w_ref[...], preferred_element_type=jnp.float32)
        + b_ref[...]
    )


def _matmul_bias(x, w, b, block_rows=1000):
    n, k = x.shape
    m = w.shape[1]
    grid = n // block_rows
    return pl.pallas_call(
        _mm_body,
        grid=(grid,),
        in_specs=[
            pl.BlockSpec((block_rows, k), lambda i: (i, 0)),
            pl.BlockSpec((k, m), lambda i: (0, 0)),
            pl.BlockSpec((1, m), lambda i: (0, 0)),
        ],
        out_specs=pl.BlockSpec((block_rows, m), lambda i: (i, 0)),
        out_shape=jax.ShapeDtypeStruct((n, m), jnp.float32),
    )(x, w, b.reshape(1, m))


def _gcn(x, src, dst, ew, W, b, n):
    sl = jnp.arange(n)
    s = jnp.concatenate([src, sl])
    d = jnp.concatenate([dst, sl])
    w = jnp.concatenate([ew, jnp.ones((n,), x.dtype)])
    deg = jax.ops.segment_sum(w, d, num_segments=n)
    dis = jnp.where(deg > 0, 1.0 / jnp.sqrt(deg), 0.0)
    norm = dis[s] * w * dis[d]
    xw = _matmul_bias(x, W, jnp.zeros((W.shape[1],), jnp.float32))
    out = jax.ops.segment_sum(xw[s] * norm[:, None], d, num_segments=n)
    return out + b


def _sag(x, src, dst, ew, npg, Wr, br, Ws):
    n = _B * npg
    aggr = jax.ops.segment_sum(x[src] * ew[:, None], dst, num_segments=n)
    score = (aggr @ Wr + br + x @ Ws).reshape(-1)
    k = npg // 2
    _, topi = jax.lax.top_k(score.reshape(_B, npg), k)
    perm = (topi + (jnp.arange(_B) * npg)[:, None]).reshape(-1)
    xp = x[perm] * jnp.tanh(score[perm])[:, None]
    newmap = jnp.full((n,), -1, dtype=jnp.int32).at[perm].set(
        jnp.arange(_B * k, dtype=jnp.int32))
    ms = newmap[src]
    md = newmap[dst]
    keep = (ms >= 0) & (md >= 0)
    nsrc = jnp.where(keep, ms, 0)
    ndst = jnp.where(keep, md, 0)
    new_ew = jnp.where(keep, ew, 0.0)
    return xp, nsrc, ndst, new_ew, perm


def _gpool(x, npg):
    xr = x.reshape(_B, npg, -1)
    return jnp.concatenate([xr.mean(axis=1), xr.max(axis=1)], axis=1)


def _lstm_dir(xs, Wih, Whh, bb, reverse):
    def step(carry, xt):
        h, c = carry
        g = xt @ Wih.T + h @ Whh.T + bb
        i, f, gg, o = jnp.split(g, 4, axis=-1)
        c = jax.nn.sigmoid(f) * c + jax.nn.sigmoid(i) * jnp.tanh(gg)
        h = jax.nn.sigmoid(o) * jnp.tanh(c)
        return (h, c), h
    z = jnp.zeros((xs.shape[1], Whh.shape[1]), xs.dtype)
    _, hs = jax.lax.scan(step, (z, z), xs, reverse=reverse)
    return hs


def _bilstm(xs, Wih, Whh, bb):
    return jnp.concatenate(
        [_lstm_dir(xs, Wih[0], Whh[0], bb[0], False),
         _lstm_dir(xs, Wih[1], Whh[1], bb[1], True)], axis=-1)


def kernel(x, edge_index, edge_attr, batch, W1, b1, s1_Wr, s1_br, s1_Ws,
           W2, b2, s2_Wr, s2_br, s2_Ws, W3, b3, s3_Wr, s3_br, s3_Ws,
           lstm_Wih0, lstm_Whh0, lstm_b0, lstm_Wih1, lstm_Whh1, lstm_b1,
           M1, mb1, M2, mb2, M3, mb3):
    src, dst = edge_index[0], edge_index[1]
    indexs = jnp.tile(jnp.arange(_N0), _B)
    x1 = jax.nn.relu(_gcn(x, src, dst, edge_attr, W1, b1, _N))
    p1, s1s, s1d, e1, perm1 = _sag(x1, src, dst, edge_attr, _N0,
                                   s1_Wr, s1_br, s1_Ws)
    indexs = indexs[perm1]
    g1 = _gpool(p1, _N0 // 2)
    x2 = jax.nn.relu(_gcn(p1, s1s, s1d, e1, W2, b2, _B * (_N0 // 2)))
    p2, s2s, s2d, e2, perm2 = _sag(x2, s1s, s1d, e1, _N0 // 2,
                                   s2_Wr, s2_br, s2_Ws)
    indexs = indexs[perm2]
    g2 = _gpool(p2, _N0 // 4)
    x3 = jax.nn.relu(_gcn(p2, s2s, s2d, e2, W3, b3, _B * (_N0 // 4)))
    p3, _, _, _, perm3 = _sag(x3, s2s, s2d, e2, _N0 // 8 * 2,
                              s3_Wr, s3_br, s3_Ws)
    indexs = indexs[perm3]
    g3 = _gpool(p3, _N0 // 8)
    readout = g1 + g2 + g3
    seq = readout.reshape(_B, 2 * _H, 1).transpose(1, 0, 2)
    lo = _bilstm(seq, lstm_Wih0, lstm_Whh0, lstm_b0)
    lo = _bilstm(lo, lstm_Wih1, lstm_Whh1, lstm_b1)
    lo = lo.transpose(1, 0, 2).mean(axis=2)
    h1 = jax.nn.relu(_matmul_bias(lo, M1, mb1, block_rows=_B))
    h2 = jax.nn.relu(_matmul_bias(h1, M2, mb2, block_rows=_B))
    logits = _matmul_bias(h2, M3, mb3, block_rows=_B)
    return jax.nn.log_softmax(logits, axis=1), indexs.reshape(_B, -1)
